# baseline (speedup 1.0000x reference)
import functools
import math

import jax
import jax.numpy as jnp
from jax import lax
from jax.experimental import pallas as pl
from jax.experimental.pallas import tpu as pltpu
from jax.experimental.pallas import tpu_sc as plsc

_L = 16


def _sc_num_workers():
    info = plsc.get_sparse_core_info()
    return info.num_cores, info.num_subcores


def kernel(input_ids, table, offsets):
    B, F = input_ids.shape
    V, D = table.shape
    NC, NS = _sc_num_workers()
    NW = NC * NS
    assert B % NW == 0
    RPW = B // NW
    IPW = RPW * F
    assert IPW % _L == 0

    GCH = 128
    assert IPW % GCH == 0
    NCHUNK = IPW // GCH
    SPG = 13
    assert NCHUNK % SPG == 0
    NGROUP = NCHUNK // SPG
    GROUP_ROWS = SPG * GCH

    ids_flat = input_ids.astype(jnp.int32).reshape(B * F)
    PERIOD = F * _L // math.gcd(F, _L)
    REP = PERIOD // F
    NPAT = PERIOD // _L
    assert IPW % PERIOD == 0
    off_pat = jnp.tile(offsets.astype(jnp.int32), REP)

    mesh = plsc.VectorSubcoreMesh(core_axis_name="c", subcore_axis_name="s")

    @functools.partial(
        pl.kernel,
        mesh=mesh,
        compiler_params=pltpu.CompilerParams(use_tc_tiling_on_sc=False),
        out_type=jax.ShapeDtypeStruct((B * F, D), jnp.float32),
        scratch_types=[
            pltpu.VMEM((IPW,), jnp.int32),
            pltpu.VMEM((PERIOD,), jnp.int32),
            pltpu.VMEM((GROUP_ROWS, D), jnp.float32),
            pltpu.SemaphoreType.DMA,
        ],
    )
    def _k(ids_hbm, table_hbm, off_hbm, out_hbm, idx_v, off_v, buf_v, sem):
        wid = lax.axis_index("s") * NC + lax.axis_index("c")
        base = wid * IPW

        pltpu.sync_copy(off_hbm, off_v)
        pltpu.sync_copy(ids_hbm.at[pl.ds(base, IPW)], idx_v)

        def add_off(j, _):
            off = off_v[pl.ds(lax.rem(j, NPAT) * _L, _L)]
            sl = pl.ds(j * _L, _L)
            idx_v[sl] = idx_v[sl] + off
            return _

        lax.fori_loop(0, IPW // _L, add_off, None)

        def group(g, _):
            gbase = g * GROUP_ROWS
            copies = []
            for i in range(SPG):
                cbase = gbase + i * GCH
                copies.append(pltpu.async_copy(
                    table_hbm.at[idx_v.at[pl.ds(cbase, GCH)]],
                    buf_v.at[pl.ds(i * GCH, GCH)],
                    sem,
                ))
            for c in copies:
                c.wait()
            pltpu.sync_copy(buf_v, out_hbm.at[pl.ds(base + gbase, GROUP_ROWS)])
            return _

        lax.fori_loop(0, NGROUP, group, None)

    out = _k(ids_flat, table, off_pat)
    return out.reshape(B, F, D)


# speedup vs baseline: 1.0003x; 1.0003x over previous
import functools
import math

import jax
import jax.numpy as jnp
from jax import lax
from jax.experimental import pallas as pl
from jax.experimental.pallas import tpu as pltpu
from jax.experimental.pallas import tpu_sc as plsc

_L = 16


def _sc_num_workers():
    info = plsc.get_sparse_core_info()
    return info.num_cores, info.num_subcores


def kernel(input_ids, table, offsets):
    B, F = input_ids.shape
    V, D = table.shape
    NC, NS = _sc_num_workers()
    NW = NC * NS
    assert B % NW == 0
    RPW = B // NW
    IPW = RPW * F
    assert IPW % _L == 0

    GCH = 128
    assert IPW % GCH == 0
    NCHUNK = IPW // GCH
    SPG = 13
    assert NCHUNK % SPG == 0
    NGROUP = NCHUNK // SPG
    GROUP_ROWS = SPG * GCH

    ids_flat = input_ids.astype(jnp.int32).reshape(B * F)
    assert V % 4 == 0
    t2 = jax.lax.optimization_barrier(table.reshape(V // 4, D * 4))
    table_lin = t2.reshape(V, D)
    PERIOD = F * _L // math.gcd(F, _L)
    REP = PERIOD // F
    NPAT = PERIOD // _L
    assert IPW % PERIOD == 0
    off_pat = jnp.tile(offsets.astype(jnp.int32), REP)

    mesh = plsc.VectorSubcoreMesh(core_axis_name="c", subcore_axis_name="s")

    @functools.partial(
        pl.kernel,
        mesh=mesh,
        compiler_params=pltpu.CompilerParams(use_tc_tiling_on_sc=False),
        out_type=jax.ShapeDtypeStruct((B * F, D), jnp.float32),
        scratch_types=[
            pltpu.VMEM((IPW,), jnp.int32),
            pltpu.VMEM((PERIOD,), jnp.int32),
            pltpu.VMEM((GROUP_ROWS, D), jnp.float32),
            pltpu.SemaphoreType.DMA,
        ],
    )
    def _k(ids_hbm, table_hbm, off_hbm, out_hbm, idx_v, off_v, buf_v, sem):
        wid = lax.axis_index("s") * NC + lax.axis_index("c")
        base = wid * IPW

        pltpu.sync_copy(off_hbm, off_v)
        pltpu.sync_copy(ids_hbm.at[pl.ds(base, IPW)], idx_v)

        def add_off(j, _):
            off = off_v[pl.ds(lax.rem(j, NPAT) * _L, _L)]
            sl = pl.ds(j * _L, _L)
            idx_v[sl] = idx_v[sl] + off
            return _

        lax.fori_loop(0, IPW // _L, add_off, None)

        def group(g, _):
            gbase = g * GROUP_ROWS
            copies = []
            for i in range(SPG):
                cbase = gbase + i * GCH
                copies.append(pltpu.async_copy(
                    table_hbm.at[idx_v.at[pl.ds(cbase, GCH)]],
                    buf_v.at[pl.ds(i * GCH, GCH)],
                    sem,
                ))
            for c in copies:
                c.wait()
            pltpu.sync_copy(buf_v, out_hbm.at[pl.ds(base + gbase, GROUP_ROWS)])
            return _

        lax.fori_loop(0, NGROUP, group, None)

    out = _k(ids_flat, table_lin, off_pat)
    return out.reshape(B, F, D)
